# Initial kernel scaffold; baseline (speedup 1.0000x reference)
import jax
import jax.numpy as jnp
from jax.experimental import pallas as pl


def kernel(x_nchw, w1_oihw, w2_oihw):
    raise NotImplementedError("write your pallas kernel here")


# speedup vs baseline: 1.9747x; 1.9747x over previous
import functools

import numpy as np
import jax
import jax.numpy as jnp
from jax import lax
from jax.experimental import pallas as pl
from jax.experimental.pallas import tpu as pltpu

_INV_SQRT2 = 0.7071067811865476


def _gelu(x):
    return 0.5 * x * (1.0 + lax.erf(x * _INV_SQRT2))


def _stack_kernel(mask_ref, x_ref, w1_ref, w2_ref, o_ref, *, n_layers, width):
    cres3 = w1_ref.shape[0]
    cres = cres3 // 3
    w1 = w1_ref[...]
    w2 = w2_ref[...]
    masks = mask_ref[...]
    m_left = masks[0:1].astype(jnp.bfloat16)
    m_right = masks[1:2].astype(jnp.bfloat16)
    m_top = masks[2:3]
    m_bot = masks[3:4]

    x0 = x_ref[0].astype(jnp.float32)

    def layer(_, x):
        xb = x.astype(jnp.bfloat16)
        u_m = jnp.roll(xb, 1, axis=1) * m_left
        u_p = jnp.roll(xb, -1, axis=1) * m_right
        s3 = jnp.concatenate([u_m, xb, u_p], axis=0)
        z = jnp.dot(w1, s3, preferred_element_type=jnp.float32)
        y = (z[cres:2 * cres]
             + m_top * jnp.roll(z[:cres], width, axis=1)
             + m_bot * jnp.roll(z[2 * cres:], -width, axis=1))
        g = _gelu(y).astype(jnp.bfloat16)
        r = _gelu(jnp.dot(w2, g, preferred_element_type=jnp.float32))
        return x + r

    o_ref[0] = lax.fori_loop(0, n_layers, layer, x0, unroll=True)


def _residual_stack(x_nchw, w1_oihw, w2_oihw, n_layers):
    N, C, H, W = x_nchw.shape
    Cres = w1_oihw.shape[0]
    HW = H * W

    x_flat = x_nchw.reshape(N, C, HW)

    w1_r = jnp.transpose(w1_oihw, (2, 0, 3, 1)).reshape(3 * Cres, 3 * C)
    w1_r = w1_r.astype(jnp.bfloat16)
    w2_m = w2_oihw[:, :, 0, 0].astype(jnp.bfloat16)

    p = np.arange(HW)
    hh, ww = p // W, p % W
    mask_np = np.zeros((8, HW), np.float32)
    mask_np[0] = ww > 0
    mask_np[1] = ww < W - 1
    mask_np[2] = hh > 0
    mask_np[3] = hh < H - 1
    mask_arr = jnp.asarray(mask_np)

    kernel_fn = functools.partial(_stack_kernel, n_layers=n_layers, width=W)

    out_flat = pl.pallas_call(
        kernel_fn,
        out_shape=jax.ShapeDtypeStruct((N, C, HW), x_nchw.dtype),
        grid=(N,),
        in_specs=[
            pl.BlockSpec((8, HW), lambda n: (0, 0)),
            pl.BlockSpec((1, C, HW), lambda n: (n, 0, 0)),
            pl.BlockSpec((3 * Cres, 3 * C), lambda n: (0, 0)),
            pl.BlockSpec((C, Cres), lambda n: (0, 0)),
        ],
        out_specs=pl.BlockSpec((1, C, HW), lambda n: (n, 0, 0)),
        compiler_params=pltpu.CompilerParams(
            dimension_semantics=("parallel",),
            vmem_limit_bytes=64 * 1024 * 1024),
    )(mask_arr, x_flat, w1_r, w2_m)

    return out_flat.reshape(N, C, H, W)


def kernel(x_nchw, w1_oihw, w2_oihw):
    return _residual_stack(x_nchw, w1_oihw, w2_oihw, 6)


# speedup vs baseline: 3.0045x; 1.5215x over previous
import functools

import numpy as np
import jax
import jax.numpy as jnp
from jax import lax
from jax.experimental import pallas as pl
from jax.experimental.pallas import tpu as pltpu

_INV_SQRT2 = 0.7071067811865476


def _gelu(x):
    return 0.5 * x * (1.0 + lax.erf(x * _INV_SQRT2))


def _stack_kernel(mask_ref, x_ref, w1_ref, w2_ref, o_ref, *, n_layers, width,
                  planes):
    cres3 = w1_ref.shape[0]
    cres = cres3 // 3
    hw = x_ref.shape[2]
    w1 = w1_ref[...]
    w2 = w2_ref[...]
    masks = mask_ref[...]
    m_left = masks[0:1].astype(jnp.bfloat16)
    m_right = masks[1:2].astype(jnp.bfloat16)
    m_top = masks[2:3]
    m_bot = masks[3:4]

    x0 = jnp.concatenate([x_ref[j] for j in range(planes)], axis=1)
    x0 = x0.astype(jnp.float32)

    def layer(_, x):
        xb = x.astype(jnp.bfloat16)
        u_m = jnp.roll(xb, 1, axis=1) * m_left
        u_p = jnp.roll(xb, -1, axis=1) * m_right
        s3 = jnp.concatenate([u_m, xb, u_p], axis=0)
        z = jnp.dot(w1, s3, preferred_element_type=jnp.float32)
        y = (z[cres:2 * cres]
             + m_top * jnp.roll(z[:cres], width, axis=1)
             + m_bot * jnp.roll(z[2 * cres:], -width, axis=1))
        g = _gelu(y).astype(jnp.bfloat16)
        r = _gelu(jnp.dot(w2, g, preferred_element_type=jnp.float32))
        return x + r

    x_out = lax.fori_loop(0, n_layers, layer, x0, unroll=True)
    for j in range(planes):
        o_ref[j] = x_out[:, j * hw:(j + 1) * hw].astype(o_ref.dtype)


def _residual_stack(x_nchw, w1_oihw, w2_oihw, n_layers):
    N, C, H, W = x_nchw.shape
    Cres = w1_oihw.shape[0]
    HW = H * W

    planes = 4
    while N % planes:
        planes //= 2
    lanes = planes * HW

    x_flat = x_nchw.reshape(N, C, HW)

    w1_r = jnp.transpose(w1_oihw, (2, 0, 3, 1)).reshape(3 * Cres, 3 * C)
    w1_r = w1_r.astype(jnp.bfloat16)
    w2_m = w2_oihw[:, :, 0, 0].astype(jnp.bfloat16)

    p = np.arange(lanes) % HW
    hh, ww = p // W, p % W
    mask_np = np.zeros((8, lanes), np.float32)
    mask_np[0] = ww > 0
    mask_np[1] = ww < W - 1
    mask_np[2] = hh > 0
    mask_np[3] = hh < H - 1
    mask_arr = jnp.asarray(mask_np)

    kernel_fn = functools.partial(_stack_kernel, n_layers=n_layers, width=W,
                                  planes=planes)

    out_flat = pl.pallas_call(
        kernel_fn,
        out_shape=jax.ShapeDtypeStruct((N, C, HW), x_nchw.dtype),
        grid=(N // planes,),
        in_specs=[
            pl.BlockSpec((8, lanes), lambda n: (0, 0)),
            pl.BlockSpec((planes, C, HW), lambda n: (n, 0, 0)),
            pl.BlockSpec((3 * Cres, 3 * C), lambda n: (0, 0)),
            pl.BlockSpec((C, Cres), lambda n: (0, 0)),
        ],
        out_specs=pl.BlockSpec((planes, C, HW), lambda n: (n, 0, 0)),
        compiler_params=pltpu.CompilerParams(
            dimension_semantics=("parallel",),
            vmem_limit_bytes=64 * 1024 * 1024),
    )(mask_arr, x_flat, w1_r, w2_m)

    return out_flat.reshape(N, C, H, W)


def kernel(x_nchw, w1_oihw, w2_oihw):
    return _residual_stack(x_nchw, w1_oihw, w2_oihw, 6)
